# baseline (speedup 1.0000x reference)
import functools

import jax
import jax.numpy as jnp
from jax import lax
from jax.experimental import pallas as pl
from jax.experimental.pallas import tpu as pltpu
from jax.experimental.pallas import tpu_sc as plsc

N_ROWS = 16384
D = 1024
L = 16
NC = 2
NS = 16
NW = NC * NS

ROWS_SC = 1024
ROWS_TC = N_ROWS - ROWS_SC
ROWS_PER_W = ROWS_SC // NW
CHUNK = 16
NCHUNK = ROWS_PER_W // CHUNK
JSLABS = D // L

BLK = 1024
NBLK = ROWS_TC // BLK
NBUF = 4


def _scores_body(a_hbm, w_hbm, out_hbm, w_v, buf_v, sc_v, sem):
    wid = lax.axis_index("s") * NC + lax.axis_index("c")
    base = ROWS_TC + wid * ROWS_PER_W
    pltpu.sync_copy(w_hbm, w_v)

    def chunk_body(c, _):
        pltpu.sync_copy(a_hbm.at[pl.ds(base + c * CHUNK, CHUNK)], buf_v)

        def jbody(j, accs):
            wj = w_v[pl.ds(j * L, L)]
            return tuple(
                accs[r] + buf_v[r, pl.ds(j * L, L)] * wj for r in range(CHUNK)
            )

        accs = lax.fori_loop(
            0, JSLABS, jbody,
            tuple(jnp.zeros((L,), jnp.float32) for _ in range(CHUNK)),
        )
        riota = lax.broadcasted_iota(jnp.int32, (L,), 0)
        svec = jnp.zeros((L,), jnp.float32)
        for r in range(L):
            svec = jnp.where(riota == r, jnp.sum(accs[r]), svec)
        sc_v[pl.ds(c * CHUNK, L)] = svec
        return 0

    lax.fori_loop(0, NCHUNK, chunk_body, 0)
    pltpu.sync_copy(sc_v, out_hbm.at[pl.ds(wid * ROWS_PER_W, ROWS_PER_W)])


_scores_sc = functools.partial(
    pl.kernel,
    out_type=jax.ShapeDtypeStruct((ROWS_SC,), jnp.float32),
    mesh=plsc.VectorSubcoreMesh(core_axis_name="c", subcore_axis_name="s"),
    compiler_params=pltpu.CompilerParams(needs_layout_passes=False),
    scratch_types=[
        pltpu.VMEM((D,), jnp.float32),
        pltpu.VMEM((CHUNK, D), jnp.float32),
        pltpu.VMEM((ROWS_PER_W,), jnp.float32),
        pltpu.SemaphoreType.DMA,
    ],
)(_scores_body)


def _tc_matvec_body(a_hbm, w_ref, o_ref, bufs, sems):
    for i in range(NBUF):
        pltpu.make_async_copy(
            a_hbm.at[pl.ds(i * BLK, BLK)], bufs.at[i], sems.at[i]
        ).start()
    w = w_ref[...]
    for i in range(NBLK):
        b = i % NBUF
        pltpu.make_async_copy(
            a_hbm.at[pl.ds(i * BLK, BLK)], bufs.at[b], sems.at[b]
        ).wait()
        o_ref[i, :] = jnp.sum(bufs[b] * w, axis=1)
        nxt = i + NBUF
        if nxt < NBLK:
            pltpu.make_async_copy(
                a_hbm.at[pl.ds(nxt * BLK, BLK)], bufs.at[b], sems.at[b]
            ).start()


def _softmax_body(tc_ref, sc_ref, o_ref):
    a = tc_ref[...]
    b = sc_ref[...]
    m = jnp.maximum(jnp.max(a), jnp.max(b))
    ea = jnp.exp(a - m)
    eb = jnp.exp(b - m)
    inv = 1.0 / (jnp.sum(ea) + jnp.sum(eb))
    o_ref[pl.ds(0, ROWS_TC)] = ea.reshape(ROWS_TC) * inv
    o_ref[pl.ds(ROWS_TC, ROWS_SC)] = eb * inv


def kernel(feature_vector, W):
    w = W.reshape(D)
    scores_sc = _scores_sc(feature_vector, w)
    scores_tc = pl.pallas_call(
        _tc_matvec_body,
        in_specs=[
            pl.BlockSpec(memory_space=pl.ANY),
            pl.BlockSpec((1, D), lambda: (0, 0)),
        ],
        out_specs=pl.BlockSpec((NBLK, BLK), lambda: (0, 0)),
        out_shape=jax.ShapeDtypeStruct((NBLK, BLK), jnp.float32),
        scratch_shapes=[
            pltpu.VMEM((NBUF, BLK, D), jnp.float32),
            pltpu.SemaphoreType.DMA((NBUF,)),
        ],
        compiler_params=pltpu.CompilerParams(skip_device_barrier=True),
    )(feature_vector, W.reshape(1, D))
    probs = pl.pallas_call(
        _softmax_body,
        out_shape=jax.ShapeDtypeStruct((N_ROWS,), jnp.float32),
    )(scores_tc, scores_sc)
    return probs.reshape(1, N_ROWS, 1)


# speedup vs baseline: 1.6651x; 1.6651x over previous
import jax
import jax.numpy as jnp
from jax.experimental import pallas as pl
from jax.experimental.pallas import tpu as pltpu

N_ROWS = 16384
D = 1024
BLK = 1024
NBLK = N_ROWS // BLK
NBUF = 4


def _fused_body(a_hbm, w_ref, o_ref, bufs, scores_v, sems):
    for i in range(NBUF):
        pltpu.make_async_copy(
            a_hbm.at[pl.ds(i * BLK, BLK)], bufs.at[i], sems.at[i]
        ).start()
    w = w_ref[...]
    for i in range(NBLK):
        b = i % NBUF
        pltpu.make_async_copy(
            a_hbm.at[pl.ds(i * BLK, BLK)], bufs.at[b], sems.at[b]
        ).wait()
        scores_v[i, :] = jnp.sum(bufs[b] * w, axis=1)
        nxt = i + NBUF
        if nxt < NBLK:
            pltpu.make_async_copy(
                a_hbm.at[pl.ds(nxt * BLK, BLK)], bufs.at[b], sems.at[b]
            ).start()
    sc = scores_v[...]
    m = jnp.max(sc)
    e = jnp.exp(sc - m)
    o_ref[...] = e * (1.0 / jnp.sum(e))


def kernel(feature_vector, W):
    probs = pl.pallas_call(
        _fused_body,
        in_specs=[
            pl.BlockSpec(memory_space=pl.ANY),
            pl.BlockSpec((1, D), lambda: (0, 0)),
        ],
        out_specs=pl.BlockSpec((NBLK, BLK), lambda: (0, 0)),
        out_shape=jax.ShapeDtypeStruct((NBLK, BLK), jnp.float32),
        scratch_shapes=[
            pltpu.VMEM((NBUF, BLK, D), jnp.float32),
            pltpu.VMEM((NBLK, BLK), jnp.float32),
            pltpu.SemaphoreType.DMA((NBUF,)),
        ],
    )(feature_vector, W.reshape(1, D))
    return probs.reshape(1, N_ROWS, 1)
